# Initial kernel scaffold; baseline (speedup 1.0000x reference)
import jax
import jax.numpy as jnp
from jax.experimental import pallas as pl


def kernel(token_ids, weight):
    raise NotImplementedError("write your pallas kernel here")


# speedup vs baseline: 1.6844x; 1.6844x over previous
import functools

import jax
import jax.numpy as jnp
from jax import lax
from jax.experimental import pallas as pl
from jax.experimental.pallas import tpu as pltpu
from jax.experimental.pallas import tpu_sc as plsc

_D = 64
_B = 16384 * 50
_NC = 2
_NS = 16
_NW = _NC * _NS
_B_PER_W = _B // _NW
_CHUNK = 128
_N_CHUNKS = _B_PER_W // _CHUNK

_mesh = plsc.VectorSubcoreMesh(core_axis_name="c", subcore_axis_name="s")


@functools.partial(
    pl.kernel,
    mesh=_mesh,
    out_type=jax.ShapeDtypeStruct((_B, _D), jnp.float32),
    scratch_types=[
        pltpu.VMEM((_N_CHUNKS, _CHUNK), jnp.int32),
        pltpu.VMEM((_CHUNK, _D), jnp.float32),
        pltpu.SemaphoreType.DMA,
    ],
    compiler_params=pltpu.CompilerParams(use_tc_tiling_on_sc=False),
)
def _emb_lookup(idx_hbm, table_hbm, out_hbm, idx_v, rows_v, sem):
    wid = lax.axis_index("s") * _NC + lax.axis_index("c")
    base = wid * _B_PER_W
    pltpu.sync_copy(idx_hbm.at[wid], idx_v)

    def body(c, carry):
        pltpu.async_copy(table_hbm.at[idx_v.at[c]], rows_v, sem).wait()
        pltpu.sync_copy(rows_v, out_hbm.at[pl.ds(base + c * _CHUNK, _CHUNK)])
        return carry

    lax.fori_loop(0, _N_CHUNKS, body, 0)


def kernel(token_ids, weight):
    idx = token_ids.reshape(_NW, _N_CHUNKS, _CHUNK)
    out = _emb_lookup(idx, weight)
    return out.reshape(token_ids.shape[0], token_ids.shape[1], _D)


# speedup vs baseline: 1.8736x; 1.1123x over previous
import functools

import jax
import jax.numpy as jnp
from jax import lax
from jax.experimental import pallas as pl
from jax.experimental.pallas import tpu as pltpu
from jax.experimental.pallas import tpu_sc as plsc

_D = 64
_B = 16384 * 50
_NC = 2
_NS = 16
_NW = _NC * _NS
_B_PER_W = _B // _NW
_CHUNK = 128
_N_CHUNKS = _B_PER_W // _CHUNK
_K = 4
_GROUP = _CHUNK * _K
_N_GROUPS = _B_PER_W // _GROUP
_N_PAIRS = _N_GROUPS // 2

_mesh = plsc.VectorSubcoreMesh(core_axis_name="c", subcore_axis_name="s")


@functools.partial(
    pl.kernel,
    mesh=_mesh,
    out_type=jax.ShapeDtypeStruct((_B, _D), jnp.float32),
    scratch_types=[
        pltpu.VMEM((_N_CHUNKS, _CHUNK), jnp.int32),
        pltpu.VMEM((_GROUP, _D), jnp.float32),
        pltpu.VMEM((_GROUP, _D), jnp.float32),
        pltpu.SemaphoreType.DMA,
        pltpu.SemaphoreType.DMA,
        pltpu.SemaphoreType.DMA,
        pltpu.SemaphoreType.DMA,
    ],
    compiler_params=pltpu.CompilerParams(use_tc_tiling_on_sc=False),
)
def _emb_lookup(idx_hbm, table_hbm, out_hbm, idx_v, buf_a, buf_b,
                sem_in_a, sem_in_b, sem_out_a, sem_out_b):
    wid = lax.axis_index("s") * _NC + lax.axis_index("c")
    base = wid * _B_PER_W
    pltpu.sync_copy(idx_hbm.at[wid], idx_v)

    def fire_gathers(g, buf, sem):
        for k in range(_K):
            pltpu.async_copy(
                table_hbm.at[idx_v.at[g * _K + k]],
                buf.at[pl.ds(k * _CHUNK, _CHUNK)],
                sem,
            )

    def wait_gathers(buf, sem):
        pltpu.make_async_copy(table_hbm.at[pl.ds(0, _GROUP)], buf, sem).wait()

    def fire_store(g, buf, sem):
        pltpu.async_copy(buf, out_hbm.at[pl.ds(base + g * _GROUP, _GROUP)], sem)

    def wait_store(buf, sem):
        pltpu.make_async_copy(buf, out_hbm.at[pl.ds(base, _GROUP)], sem).wait()

    fire_gathers(0, buf_a, sem_in_a)

    def pair(i, carry):
        g = 2 * i
        @pl.when(i > 0)
        def _():
            wait_store(buf_b, sem_out_b)
        fire_gathers(g + 1, buf_b, sem_in_b)

        wait_gathers(buf_a, sem_in_a)
        fire_store(g, buf_a, sem_out_a)
        wait_store(buf_a, sem_out_a)
        @pl.when(i < _N_PAIRS - 1)
        def _():
            fire_gathers(g + 2, buf_a, sem_in_a)

        wait_gathers(buf_b, sem_in_b)
        fire_store(g + 1, buf_b, sem_out_b)
        return carry

    lax.fori_loop(0, _N_PAIRS, pair, 0)

    wait_store(buf_b, sem_out_b)


def kernel(token_ids, weight):
    idx = token_ids.reshape(_NW, _N_CHUNKS, _CHUNK)
    out = _emb_lookup(idx, weight)
    return out.reshape(token_ids.shape[0], token_ids.shape[1], _D)


# speedup vs baseline: 1.8755x; 1.0010x over previous
import functools

import jax
import jax.numpy as jnp
from jax import lax
from jax.experimental import pallas as pl
from jax.experimental.pallas import tpu as pltpu
from jax.experimental.pallas import tpu_sc as plsc

_D = 64
_B = 16384 * 50
_NC = 2
_NS = 16
_NW = _NC * _NS
_B_PER_W = _B // _NW
_CHUNK = 512
_N_CHUNKS = _B_PER_W // _CHUNK
_K = 1
_GROUP = _CHUNK * _K
_N_GROUPS = _B_PER_W // _GROUP
_N_PAIRS = _N_GROUPS // 2

_mesh = plsc.VectorSubcoreMesh(core_axis_name="c", subcore_axis_name="s")


@functools.partial(
    pl.kernel,
    mesh=_mesh,
    out_type=jax.ShapeDtypeStruct((_B, _D), jnp.float32),
    scratch_types=[
        pltpu.VMEM((_N_CHUNKS, _CHUNK), jnp.int32),
        pltpu.VMEM((_GROUP, _D), jnp.float32),
        pltpu.VMEM((_GROUP, _D), jnp.float32),
        pltpu.SemaphoreType.DMA,
        pltpu.SemaphoreType.DMA,
        pltpu.SemaphoreType.DMA,
        pltpu.SemaphoreType.DMA,
    ],
    compiler_params=pltpu.CompilerParams(use_tc_tiling_on_sc=False),
)
def _emb_lookup(idx_hbm, table_hbm, out_hbm, idx_v, buf_a, buf_b,
                sem_in_a, sem_in_b, sem_out_a, sem_out_b):
    wid = lax.axis_index("s") * _NC + lax.axis_index("c")
    base = wid * _B_PER_W
    pltpu.sync_copy(idx_hbm.at[wid], idx_v)

    def fire_gathers(g, buf, sem):
        for k in range(_K):
            pltpu.async_copy(
                table_hbm.at[idx_v.at[g * _K + k]],
                buf.at[pl.ds(k * _CHUNK, _CHUNK)],
                sem,
            )

    def wait_gathers(buf, sem):
        pltpu.make_async_copy(table_hbm.at[pl.ds(0, _GROUP)], buf, sem).wait()

    def fire_store(g, buf, sem):
        pltpu.async_copy(buf, out_hbm.at[pl.ds(base + g * _GROUP, _GROUP)], sem)

    def wait_store(buf, sem):
        pltpu.make_async_copy(buf, out_hbm.at[pl.ds(base, _GROUP)], sem).wait()

    fire_gathers(0, buf_a, sem_in_a)

    def pair(i, carry):
        g = 2 * i
        @pl.when(i > 0)
        def _():
            wait_store(buf_b, sem_out_b)
        fire_gathers(g + 1, buf_b, sem_in_b)

        wait_gathers(buf_a, sem_in_a)
        fire_store(g, buf_a, sem_out_a)
        wait_store(buf_a, sem_out_a)
        @pl.when(i < _N_PAIRS - 1)
        def _():
            fire_gathers(g + 2, buf_a, sem_in_a)

        wait_gathers(buf_b, sem_in_b)
        fire_store(g + 1, buf_b, sem_out_b)
        return carry

    lax.fori_loop(0, _N_PAIRS, pair, 0)

    wait_store(buf_b, sem_out_b)


def kernel(token_ids, weight):
    idx = token_ids.reshape(_NW, _N_CHUNKS, _CHUNK)
    out = _emb_lookup(idx, weight)
    return out.reshape(token_ids.shape[0], token_ids.shape[1], _D)


# speedup vs baseline: 3.2898x; 1.7541x over previous
import functools

import jax
import jax.numpy as jnp
from jax import lax
from jax.experimental import pallas as pl
from jax.experimental.pallas import tpu as pltpu
from jax.experimental.pallas import tpu_sc as plsc

_V = 1_000_000
_D = 64
_NB = 16384
_S = 50
_NC = 2
_NS = 16
_D_PER_C = _D // _NC
_B_PER_T = _NB // _NS
_SG = 5
_NG = _S // _SG
_IDXW = _S * _B_PER_T
_GW = _SG * _B_PER_T

_mesh = plsc.VectorSubcoreMesh(core_axis_name="c", subcore_axis_name="s")


@functools.partial(
    pl.kernel,
    mesh=_mesh,
    out_type=jax.ShapeDtypeStruct((_S, _D, _NB), jnp.float32),
    scratch_types=[
        pltpu.VMEM((1, _IDXW), jnp.int32),
        pltpu.VMEM((1, _GW), jnp.float32),
        pltpu.VMEM((1, _GW), jnp.float32),
        pltpu.VMEM_SHARED((1, _V), jnp.float32),
        pltpu.SemaphoreType.DMA,
        pltpu.SemaphoreType.DMA,
        pltpu.SemaphoreType.DMA,
        pltpu.SemaphoreType.DMA,
    ],
)
def _emb_lookup(idx_hbm, wt_hbm, out_hbm, idx_v, gbuf0, gbuf1,
                row, sem_sa, sem_g, sem_t0, sem_t1):
    cid = lax.axis_index("c")
    tid = lax.axis_index("s")
    d_base = cid * _D_PER_C
    b0 = tid * _B_PER_T

    gbufs = (gbuf0, gbuf1)
    sem_ts = (sem_t0, sem_t1)

    for s in range(_S):
        pltpu.async_copy(idx_hbm.at[:, pl.ds(s * _NB + b0, _B_PER_T)],
                         idx_v.at[:, pl.ds(s * _B_PER_T, _B_PER_T)],
                         sem_g)
    pltpu.make_async_copy(idx_hbm.at[:, pl.ds(0, _IDXW)], idx_v,
                          sem_g).wait()

    def stage(j, row, sem):
        pltpu.async_copy(wt_hbm.at[pl.ds(d_base + j, 1), :], row, sem)

    def wait_stage(row, sem):
        pltpu.make_async_copy(wt_hbm.at[pl.ds(0, 1), :], row, sem).wait()

    def body(j, carry):
        @pl.when(tid == 0)
        def _():
            stage(j, row, sem_sa)
            wait_stage(row, sem_sa)
        plsc.subcore_barrier()
        dd = d_base + j
        for g in range(_NG):
            h = g % 2
            @pl.when((j >= 1) | (g >= 2))
            def _():
                pltpu.make_async_copy(
                    wt_hbm.at[pl.ds(0, 1), pl.ds(0, _GW)],
                    gbufs[h], sem_ts[h]).wait()
            for s in range(_SG):
                pltpu.async_copy(
                    row.at[idx_v.at[:, pl.ds((g * _SG + s) * _B_PER_T,
                                             _B_PER_T)]],
                    gbufs[h].at[:, pl.ds(s * _B_PER_T, _B_PER_T)],
                    sem_g)
            pltpu.make_async_copy(
                wt_hbm.at[pl.ds(0, 1), pl.ds(0, _GW)],
                gbufs[h], sem_g).wait()
            for s in range(_SG):
                pltpu.async_copy(
                    gbufs[h].at[:, pl.ds(s * _B_PER_T, _B_PER_T)],
                    out_hbm.at[pl.ds(g * _SG + s, 1), dd,
                               pl.ds(b0, _B_PER_T)],
                    sem_ts[h])

    def body_with_tail_barrier(j, carry):
        body(j, carry)
        plsc.subcore_barrier()
        return carry

    lax.fori_loop(0, _D_PER_C, body_with_tail_barrier, 0)

    for h in range(2):
        pltpu.make_async_copy(
            wt_hbm.at[pl.ds(0, 1), pl.ds(0, _GW)],
            gbufs[h], sem_ts[h]).wait()


def kernel(token_ids, weight):
    idx_flat = token_ids.T.reshape(1, _S * _NB)
    out_t = _emb_lookup(idx_flat, weight.T)
    return jnp.transpose(out_t, (2, 0, 1))


# speedup vs baseline: 3.2931x; 1.0010x over previous
import functools

import jax
import jax.numpy as jnp
from jax import lax
from jax.experimental import pallas as pl
from jax.experimental.pallas import tpu as pltpu
from jax.experimental.pallas import tpu_sc as plsc

_V = 1_000_000
_D = 64
_NB = 16384
_S = 50
_NC = 2
_NS = 16
_D_PER_C = _D // _NC
_B_PER_T = _NB // _NS
_SG = 5
_NG = _S // _SG
_IDXW = _S * _B_PER_T
_GW = _SG * _B_PER_T

_mesh = plsc.VectorSubcoreMesh(core_axis_name="c", subcore_axis_name="s")


@functools.partial(
    pl.kernel,
    mesh=_mesh,
    out_type=jax.ShapeDtypeStruct((_S, _D, _NB), jnp.float32),
    scratch_types=[
        pltpu.VMEM((1, _IDXW), jnp.int32),
        pltpu.VMEM((1, _GW), jnp.float32),
        pltpu.VMEM((1, _GW), jnp.float32),
        pltpu.VMEM_SHARED((1, _V), jnp.float32),
        pltpu.SemaphoreType.DMA,
        pltpu.SemaphoreType.DMA,
        pltpu.SemaphoreType.DMA,
        pltpu.SemaphoreType.DMA,
    ],
)
def _emb_lookup(idx_hbm, wt_hbm, out_hbm, idx_v, gbuf0, gbuf1,
                row, sem_sa, sem_g, sem_t0, sem_t1):
    cid = lax.axis_index("c")
    tid = lax.axis_index("s")
    d_base = cid * _D_PER_C
    b0 = tid * _B_PER_T

    gbufs = (gbuf0, gbuf1)
    sem_ts = (sem_t0, sem_t1)

    for s in range(_S):
        pltpu.async_copy(idx_hbm.at[:, pl.ds(s * _NB + b0, _B_PER_T)],
                         idx_v.at[:, pl.ds(s * _B_PER_T, _B_PER_T)],
                         sem_g)
    pltpu.make_async_copy(idx_hbm.at[:, pl.ds(0, _IDXW)], idx_v,
                          sem_g).wait()

    def stage(j, row, sem):
        pltpu.async_copy(wt_hbm.at[pl.ds(d_base + j, 1), :], row, sem)

    def wait_stage(row, sem):
        pltpu.make_async_copy(wt_hbm.at[pl.ds(0, 1), :], row, sem).wait()

    def body(j, carry):
        @pl.when(tid == 0)
        def _():
            stage(j, row, sem_sa)
            wait_stage(row, sem_sa)
        plsc.subcore_barrier()
        dd = d_base + j
        for g in range(_NG):
            h = g % 2
            @pl.when((j >= 1) | (g >= 2))
            def _():
                pltpu.make_async_copy(
                    wt_hbm.at[pl.ds(0, 1), pl.ds(0, _GW)],
                    gbufs[h], sem_ts[h]).wait()
            pltpu.async_copy(
                row.at[idx_v.at[:, pl.ds(g * _GW, _GW)]],
                gbufs[h], sem_g).wait()
            pltpu.async_copy(
                gbufs[h].reshape(_SG, _B_PER_T),
                out_hbm.at[pl.ds(g * _SG, _SG), dd,
                           pl.ds(b0, _B_PER_T)],
                sem_ts[h])

    def body_with_tail_barrier(j, carry):
        body(j, carry)
        plsc.subcore_barrier()
        return carry

    lax.fori_loop(0, _D_PER_C, body_with_tail_barrier, 0)

    for h in range(2):
        pltpu.make_async_copy(
            wt_hbm.at[pl.ds(0, 1), pl.ds(0, _GW)],
            gbufs[h], sem_ts[h]).wait()


def kernel(token_ids, weight):
    idx_flat = token_ids.T.reshape(1, _S * _NB)
    out_t = _emb_lookup(idx_flat, weight.T)
    return jnp.transpose(out_t, (2, 0, 1))


# speedup vs baseline: 3.3579x; 1.0197x over previous
import functools

import jax
import jax.numpy as jnp
from jax import lax
from jax.experimental import pallas as pl
from jax.experimental.pallas import tpu as pltpu
from jax.experimental.pallas import tpu_sc as plsc

_V = 1_000_000
_D = 64
_NB = 16384
_S = 50
_NC = 2
_NS = 16
_D_PER_C = _D // _NC
_B_PER_T = _NB // _NS
_SG = 5
_NG = _S // _SG
_IDXW = _S * _B_PER_T
_GW = _SG * _B_PER_T

_mesh = plsc.VectorSubcoreMesh(core_axis_name="c", subcore_axis_name="s")


@functools.partial(
    pl.kernel,
    mesh=_mesh,
    out_type=jax.ShapeDtypeStruct((_S, _D, _NB), jnp.float32),
    scratch_types=[
        pltpu.VMEM((1, _IDXW), jnp.int32),
        pltpu.VMEM((1, _GW), jnp.float32),
        pltpu.VMEM((1, _GW), jnp.float32),
        pltpu.VMEM_SHARED((1, _V), jnp.float32),
        pltpu.SemaphoreType.DMA,
        pltpu.SemaphoreType.DMA,
        pltpu.SemaphoreType.DMA,
        pltpu.SemaphoreType.DMA,
        pltpu.SemaphoreType.DMA,
    ],
)
def _emb_lookup(idx_hbm, wt_hbm, out_hbm, idx_v, gbuf0, gbuf1,
                row, sem_sa, sem_g, sem_g1, sem_t0, sem_t1):
    cid = lax.axis_index("c")
    tid = lax.axis_index("s")
    d_base = cid * _D_PER_C
    b0 = tid * _B_PER_T

    gbufs = (gbuf0, gbuf1)
    sem_ts = (sem_t0, sem_t1)

    for s in range(_S):
        pltpu.async_copy(idx_hbm.at[:, pl.ds(s * _NB + b0, _B_PER_T)],
                         idx_v.at[:, pl.ds(s * _B_PER_T, _B_PER_T)],
                         sem_g)
    pltpu.make_async_copy(idx_hbm.at[:, pl.ds(0, _IDXW)], idx_v,
                          sem_g).wait()

    def stage(j, row, sem):
        pltpu.async_copy(wt_hbm.at[pl.ds(d_base + j, 1), :], row, sem)

    def wait_stage(row, sem):
        pltpu.make_async_copy(wt_hbm.at[pl.ds(0, 1), :], row, sem).wait()

    def body(j, carry):
        @pl.when(tid == 0)
        def _():
            stage(j, row, sem_sa)
            wait_stage(row, sem_sa)
        plsc.subcore_barrier()
        dd = d_base + j
        sem_gs = (sem_g, sem_g1)

        def drain_store(h):
            pltpu.make_async_copy(
                wt_hbm.at[pl.ds(0, 1), pl.ds(0, _GW)],
                gbufs[h], sem_ts[h]).wait()

        def fire_gather(g, h):
            pltpu.async_copy(
                row.at[idx_v.at[:, pl.ds(g * _GW, _GW)]],
                gbufs[h], sem_gs[h])

        def wait_gather(h):
            pltpu.make_async_copy(
                wt_hbm.at[pl.ds(0, 1), pl.ds(0, _GW)],
                gbufs[h], sem_gs[h]).wait()

        def fire_store(g, h):
            pltpu.async_copy(
                gbufs[h].reshape(_SG, _B_PER_T),
                out_hbm.at[pl.ds(g * _SG, _SG), dd,
                           pl.ds(b0, _B_PER_T)],
                sem_ts[h])

        @pl.when(j >= 1)
        def _():
            drain_store(0)
        fire_gather(0, 0)
        for g in range(_NG):
            h = g % 2
            if g + 1 < _NG:
                nh = 1 - h
                if g + 1 >= 2:
                    drain_store(nh)
                else:
                    @pl.when(j >= 1)
                    def _():
                        drain_store(nh)
                fire_gather(g + 1, nh)
            wait_gather(h)
            fire_store(g, h)

    def body_with_tail_barrier(j, carry):
        body(j, carry)
        plsc.subcore_barrier()
        return carry

    lax.fori_loop(0, _D_PER_C, body_with_tail_barrier, 0)

    for h in range(2):
        pltpu.make_async_copy(
            wt_hbm.at[pl.ds(0, 1), pl.ds(0, _GW)],
            gbufs[h], sem_ts[h]).wait()


def kernel(token_ids, weight):
    idx_flat = token_ids.T.reshape(1, _S * _NB)
    out_t = _emb_lookup(idx_flat, weight.T)
    return jnp.transpose(out_t, (2, 0, 1))
